# baseline (speedup 1.0000x reference)
import jax
import jax.numpy as jnp
from jax import lax
from jax.experimental import pallas as pl
from jax.experimental.pallas import tpu as pltpu
from jax.experimental.pallas import tpu_sc as plsc

MIN_VALUE = -4.0
MAX_VALUE = 4.0
N_BINS = 2048
EMBED_DIMS = 64
N_ROWS = 4096
N_COLS = 100
LANES = 16

K_BLOCKS = 2
K_PER = EMBED_DIMS // K_BLOCKS
I_BLOCKS = 16
I_PER = N_ROWS // I_BLOCKS
GROUPS = I_PER // LANES

INV_STEP = (N_BINS - 2) / (MAX_VALUE - MIN_VALUE)
MAX_K = float(N_BINS - 3)


def _sc_body_final(xt_hbm, bins_hbm, tabt_hbm, out_hbm, bins_v, tab_v, x_v,
                   stage_v, sem0, sem1):
    wid = lax.axis_index("s") * 2 + lax.axis_index("c")
    ib = wid // K_BLOCKS
    kb = wid % K_BLOCKS
    i0 = ib * I_PER
    k0 = kb * K_PER

    cpb = pltpu.async_copy(bins_hbm, bins_v.at[pl.ds(0, N_BINS - 1)], sem0)
    cpt = pltpu.async_copy(tabt_hbm.at[pl.ds(k0, K_PER), :], tab_v, sem1)
    cpx = pltpu.async_copy(xt_hbm.at[:, pl.ds(i0, I_PER)], x_v, sem0)
    cpb.wait()
    cpt.wait()
    cpx.wait()

    sems = (sem0, sem1)

    def column(j, p):
        @plsc.parallel_loop(0, GROUPS, unroll=4)
        def group(g):
            xv = x_v[j, pl.ds(g * LANES, LANES)]
            t = (xv - MIN_VALUE) * INV_STEP
            t = jnp.minimum(jnp.maximum(t, 0.0), MAX_K)
            kk = t.astype(jnp.int32)
            bk = plsc.load_gather(bins_v, [kk])
            bk1 = plsc.load_gather(bins_v, [kk + 1])
            idx = kk + (xv > bk).astype(jnp.int32) + (xv > bk1).astype(jnp.int32)
            for kq in range(K_PER // 8):
                vals = [
                    plsc.load_gather(
                        tab_v, [jnp.full((LANES,), kq * 8 + t_, jnp.int32), idx])
                    for t_ in range(8)
                ]
                for t_ in range(8):
                    stage_v[p, kq * 8 + t_, pl.ds(g * LANES, LANES)] = vals[t_]

    def write(j, p):
        return pltpu.async_copy(
            stage_v.at[p], out_hbm.at[j, pl.ds(k0, K_PER), pl.ds(i0, I_PER)],
            sems[p])

    column(0, 0)
    cp0 = write(0, 0)
    column(1, 1)
    cp1 = write(1, 1)

    def step(j2, _):
        j = j2 * 2 + 2
        cp0.wait()
        column(j, 0)
        write(j, 0)
        cp1.wait()
        column(j + 1, 1)
        write(j + 1, 1)
        return 0

    lax.fori_loop(0, (N_COLS - 2) // 2, step, 0)
    cp0.wait()
    cp1.wait()


@jax.jit
def _embed_lookup(xt, bins, tabt):
    mesh = plsc.VectorSubcoreMesh(core_axis_name="c", subcore_axis_name="s")
    return pl.kernel(
        _sc_body_final,
        out_type=jax.ShapeDtypeStruct((N_COLS, EMBED_DIMS, N_ROWS), jnp.float32),
        mesh=mesh,
        compiler_params=pltpu.CompilerParams(
            needs_layout_passes=False, use_tc_tiling_on_sc=True
        ),
        scratch_types=[
            pltpu.VMEM((N_BINS,), jnp.float32),
            pltpu.VMEM((K_PER, N_BINS), jnp.float32),
            pltpu.VMEM((N_COLS, I_PER), jnp.float32),
            pltpu.VMEM((2, K_PER, I_PER), jnp.float32),
            pltpu.SemaphoreType.DMA,
            pltpu.SemaphoreType.DMA,
        ],
    )(xt, bins, tabt)


def kernel(x, bins, table):
    xt = x.T
    tabt = table.T
    out_t = _embed_lookup(xt, bins, tabt)
    return jnp.transpose(out_t, (2, 0, 1))


# speedup vs baseline: 1.0229x; 1.0229x over previous
import jax
import jax.numpy as jnp
from jax import lax
from jax.experimental import pallas as pl
from jax.experimental.pallas import tpu as pltpu
from jax.experimental.pallas import tpu_sc as plsc

MIN_VALUE = -4.0
MAX_VALUE = 4.0
N_BINS = 2048
EMBED_DIMS = 64
N_ROWS = 4096
N_COLS = 100
LANES = 16

K_BLOCKS = 2
K_PER = EMBED_DIMS // K_BLOCKS
I_BLOCKS = 16
I_PER = N_ROWS // I_BLOCKS
GROUPS = I_PER // LANES

INV_STEP = (N_BINS - 2) / (MAX_VALUE - MIN_VALUE)
MAX_K = float(N_BINS - 3)


def _sc_body_final(xt_hbm, bins_hbm, tabt_hbm, out_hbm, bins_v, tab_v, x_v,
                   idx_v, stage_v, sem0, sem1):
    wid = lax.axis_index("s") * 2 + lax.axis_index("c")
    ib = wid // K_BLOCKS
    kb = wid % K_BLOCKS
    i0 = ib * I_PER
    k0 = kb * K_PER

    cpb = pltpu.async_copy(bins_hbm, bins_v.at[pl.ds(0, N_BINS - 1)], sem0)
    cpt = pltpu.async_copy(tabt_hbm.at[pl.ds(k0, K_PER), :], tab_v, sem1)
    cpx = pltpu.async_copy(xt_hbm.at[:, pl.ds(i0, I_PER)], x_v, sem0)
    cpb.wait()
    cpt.wait()
    cpx.wait()

    sems = (sem0, sem1)

    def column(j, p):
        @plsc.parallel_loop(0, GROUPS, unroll=4)
        def bucket(g):
            xv = x_v[j, pl.ds(g * LANES, LANES)]
            t = (xv - MIN_VALUE) * INV_STEP
            t = jnp.minimum(jnp.maximum(t, 0.0), MAX_K)
            kk = t.astype(jnp.int32)
            bk = plsc.load_gather(bins_v, [kk])
            bk1 = plsc.load_gather(bins_v, [kk + 1])
            idx = kk + (xv > bk).astype(jnp.int32) + (xv > bk1).astype(jnp.int32)
            idx_v[pl.ds(g * LANES, LANES)] = idx

        @plsc.parallel_loop(0, GROUPS, unroll=4)
        def group(g):
            idx = idx_v[pl.ds(g * LANES, LANES)]
            for kq in range(K_PER // 8):
                vals = [
                    plsc.load_gather(
                        tab_v, [jnp.full((LANES,), kq * 8 + t_, jnp.int32), idx])
                    for t_ in range(8)
                ]
                for t_ in range(8):
                    stage_v[p, kq * 8 + t_, pl.ds(g * LANES, LANES)] = vals[t_]

    def write(j, p):
        return pltpu.async_copy(
            stage_v.at[p], out_hbm.at[j, pl.ds(k0, K_PER), pl.ds(i0, I_PER)],
            sems[p])

    column(0, 0)
    cp0 = write(0, 0)
    column(1, 1)
    cp1 = write(1, 1)

    def step(j2, _):
        j = j2 * 2 + 2
        cp0.wait()
        column(j, 0)
        write(j, 0)
        cp1.wait()
        column(j + 1, 1)
        write(j + 1, 1)
        return 0

    lax.fori_loop(0, (N_COLS - 2) // 2, step, 0)
    cp0.wait()
    cp1.wait()


@jax.jit
def _embed_lookup(xt, bins, tabt):
    mesh = plsc.VectorSubcoreMesh(core_axis_name="c", subcore_axis_name="s")
    return pl.kernel(
        _sc_body_final,
        out_type=jax.ShapeDtypeStruct((N_COLS, EMBED_DIMS, N_ROWS), jnp.float32),
        mesh=mesh,
        compiler_params=pltpu.CompilerParams(
            needs_layout_passes=False, use_tc_tiling_on_sc=True
        ),
        scratch_types=[
            pltpu.VMEM((N_BINS,), jnp.float32),
            pltpu.VMEM((K_PER, N_BINS), jnp.float32),
            pltpu.VMEM((N_COLS, I_PER), jnp.float32),
            pltpu.VMEM((I_PER,), jnp.int32),
            pltpu.VMEM((2, K_PER, I_PER), jnp.float32),
            pltpu.SemaphoreType.DMA,
            pltpu.SemaphoreType.DMA,
        ],
    )(xt, bins, tabt)


def kernel(x, bins, table):
    xt = x.T
    tabt = table.T
    out_t = _embed_lookup(xt, bins, tabt)
    return jnp.transpose(out_t, (2, 0, 1))
